# Initial kernel scaffold; baseline (speedup 1.0000x reference)
import jax
import jax.numpy as jnp
from jax.experimental import pallas as pl


def kernel(hidden_states, gate_w, Wg, Wu, Wd):
    raise NotImplementedError("write your pallas kernel here")


# speedup vs baseline: 1.5324x; 1.5324x over previous
import jax
import jax.numpy as jnp
from jax import lax
from jax.experimental import pallas as pl
from jax.experimental.pallas import tpu as pltpu
from jax.experimental.pallas import tpu_sc as plsc

B, S, H, E, F = 1, 2048, 2048, 16, 768
BLK = 128
NUM_BLOCKS = 48
NSLOTS = NUM_BLOCKS * BLK
NW = 32
SLOTS_PER_W = NSLOTS // NW
TOK_PER_W = S // NW
GCHUNK = 32
CCHUNK = 16


def _router_kernel(x_ref, gw_ref, logits_ref, w_ref, pos_ref, be_ref):
    x = x_ref[...]
    gw = gw_ref[...]
    logits = lax.dot_general(x, gw, (((1,), (1,)), ((), ())),
                             preferred_element_type=jnp.float32)
    logits_ref[...] = logits

    col = lax.broadcasted_iota(jnp.int32, (S, E), 1).astype(jnp.float32)
    big = jnp.float32(1e9)
    neg = jnp.float32(-1e30)

    m1 = jnp.max(logits, axis=1, keepdims=True)
    a1 = jnp.min(jnp.where(logits == m1, col, big), axis=1, keepdims=True)
    l2 = jnp.where(col == a1, neg, logits)
    m2 = jnp.max(l2, axis=1, keepdims=True)
    a2 = jnp.min(jnp.where(l2 == m2, col, big), axis=1, keepdims=True)

    w0 = 1.0 / (1.0 + jnp.exp(m2 - m1))
    w_ref[:, 0:1] = w0
    w_ref[:, 1:2] = 1.0 - w0

    oh1 = (col == a1).astype(jnp.float32)
    oh2 = (col == a2).astype(jnp.float32)
    oh12 = oh1 + oh2

    counts = jnp.sum(oh12, axis=0, keepdims=True)
    blocks = jnp.floor((counts + (BLK - 1)) * (1.0 / BLK))

    r16 = lax.broadcasted_iota(jnp.int32, (E, E), 0)
    c16 = lax.broadcasted_iota(jnp.int32, (E, E), 1)
    tri_e = (r16 < c16).astype(jnp.float32)
    start_rows = lax.dot_general(blocks, tri_e, (((1,), (0,)), ((), ())),
                                 preferred_element_type=jnp.float32) * BLK
    end_rows = start_rows + blocks * BLK

    rt = lax.broadcasted_iota(jnp.int32, (S, S), 0)
    ct = lax.broadcasted_iota(jnp.int32, (S, S), 1)
    tri_t = (ct < rt).astype(jnp.float32)
    c_excl = lax.dot_general(tri_t, oh12, (((1,), (0,)), ((), ())),
                             preferred_element_type=jnp.float32)

    rank1 = jnp.sum(c_excl * oh1, axis=1, keepdims=True)
    rank2 = jnp.sum(c_excl * oh2, axis=1, keepdims=True)
    start1 = jnp.sum(start_rows * oh1, axis=1, keepdims=True)
    start2 = jnp.sum(start_rows * oh2, axis=1, keepdims=True)
    pos_ref[:, 0:1] = (start1 + rank1).astype(jnp.int32)
    pos_ref[:, 1:2] = (start2 + rank2).astype(jnp.int32)

    bidx = lax.broadcasted_iota(jnp.int32, (NUM_BLOCKS, E), 0).astype(jnp.float32) * BLK
    endb = jnp.broadcast_to(end_rows, (NUM_BLOCKS, E))
    be = jnp.sum((endb <= bidx).astype(jnp.float32), axis=1, keepdims=True)
    be_ref[...] = jnp.minimum(be, float(E - 1)).astype(jnp.int32)


def _run_router(x, gate_w):
    return pl.pallas_call(
        _router_kernel,
        out_shape=(
            jax.ShapeDtypeStruct((S, E), jnp.float32),
            jax.ShapeDtypeStruct((S, 2), jnp.float32),
            jax.ShapeDtypeStruct((S, 2), jnp.int32),
            jax.ShapeDtypeStruct((NUM_BLOCKS, 1), jnp.int32),
        ),
    )(x, gate_w)


def _dispatch_kernel(pos0_hbm, pos1_hbm, x_hbm, xs_hbm,
                     p0_v, p1_v, tfs_v, rows_v, sem):
    wid = lax.axis_index("s") * 2 + lax.axis_index("c")
    lo = pl.multiple_of(wid * SLOTS_PER_W, SLOTS_PER_W)

    pltpu.sync_copy(pos0_hbm, p0_v)
    pltpu.sync_copy(pos1_hbm, p1_v)

    zeros = jnp.zeros((16,), jnp.int32)

    def init_body(i, carry):
        tfs_v[pl.ds(i * 16, 16)] = zeros
        return carry

    lax.fori_loop(0, SLOTS_PER_W // 16, init_body, 0)

    iota = lax.iota(jnp.int32, 16)

    def scan_body(p_v):
        def body(i, carry):
            pv = p_v[pl.ds(i * 16, 16)]
            tok = iota + i * 16
            loc = pv - lo
            m = (loc >= 0) & (loc < SLOTS_PER_W)
            plsc.store_scatter(tfs_v, [loc], tok, mask=m)
            return carry
        lax.fori_loop(0, S // 16, body, 0)

    scan_body(p0_v)
    scan_body(p1_v)

    for c in range(SLOTS_PER_W // GCHUNK):
        idx = tfs_v.at[pl.ds(c * GCHUNK, GCHUNK)]
        pltpu.async_copy(x_hbm.at[idx], rows_v, sem).wait()
        dst = pl.multiple_of(lo + c * GCHUNK, GCHUNK)
        pltpu.sync_copy(rows_v, xs_hbm.at[pl.ds(dst, GCHUNK)])


def _run_dispatch(pos0, pos1, x):
    mesh = plsc.VectorSubcoreMesh(core_axis_name="c", subcore_axis_name="s",
                                  num_cores=2, num_subcores=16)
    return pl.kernel(
        _dispatch_kernel,
        out_type=jax.ShapeDtypeStruct((NSLOTS, H), jnp.float32),
        mesh=mesh,
        compiler_params=pltpu.CompilerParams(needs_layout_passes=False),
        scratch_types=[
            pltpu.VMEM((S,), jnp.int32),
            pltpu.VMEM((S,), jnp.int32),
            pltpu.VMEM((SLOTS_PER_W,), jnp.int32),
            pltpu.VMEM((GCHUNK, H), jnp.float32),
            pltpu.SemaphoreType.DMA,
        ],
    )(pos0, pos1, x)


def _ffn_kernel(be_ref, xs_ref, wg_ref, wu_ref, wd_ref, y_ref):
    del be_ref
    x = xs_ref[...]
    g = lax.dot_general(x, wg_ref[0], (((1,), (1,)), ((), ())),
                        preferred_element_type=jnp.float32)
    u = lax.dot_general(x, wu_ref[0], (((1,), (1,)), ((), ())),
                        preferred_element_type=jnp.float32)
    h = g / (1.0 + jnp.exp(-g)) * u
    y_ref[...] = lax.dot_general(h, wd_ref[0], (((1,), (1,)), ((), ())),
                                 preferred_element_type=jnp.float32)


def _run_ffn(be, xs, wg, wu, wd):
    grid_spec = pltpu.PrefetchScalarGridSpec(
        num_scalar_prefetch=1,
        grid=(NUM_BLOCKS,),
        in_specs=[
            pl.BlockSpec((BLK, H), lambda i, be: (i, 0)),
            pl.BlockSpec((1, F, H), lambda i, be: (be[i], 0, 0)),
            pl.BlockSpec((1, F, H), lambda i, be: (be[i], 0, 0)),
            pl.BlockSpec((1, H, F), lambda i, be: (be[i], 0, 0)),
        ],
        out_specs=pl.BlockSpec((BLK, H), lambda i, be: (i, 0)),
    )
    return pl.pallas_call(
        _ffn_kernel,
        grid_spec=grid_spec,
        out_shape=jax.ShapeDtypeStruct((NSLOTS, H), jnp.float32),
    )(be, xs, wg, wu, wd)


def _combine_kernel(pos0_hbm, pos1_hbm, y_hbm, yg0_hbm, yg1_hbm,
                    idx_v, rows_v, sem):
    wid = lax.axis_index("s") * 2 + lax.axis_index("c")
    for c in range(TOK_PER_W // CCHUNK):
        tbase = pl.multiple_of(wid * TOK_PER_W + c * CCHUNK, CCHUNK)
        pltpu.sync_copy(pos0_hbm.at[pl.ds(tbase, CCHUNK)], idx_v)
        pltpu.async_copy(y_hbm.at[idx_v], rows_v, sem).wait()
        pltpu.sync_copy(rows_v, yg0_hbm.at[pl.ds(tbase, CCHUNK)])
        pltpu.sync_copy(pos1_hbm.at[pl.ds(tbase, CCHUNK)], idx_v)
        pltpu.async_copy(y_hbm.at[idx_v], rows_v, sem).wait()
        pltpu.sync_copy(rows_v, yg1_hbm.at[pl.ds(tbase, CCHUNK)])


def _run_combine(pos0, pos1, y):
    mesh = plsc.VectorSubcoreMesh(core_axis_name="c", subcore_axis_name="s",
                                  num_cores=2, num_subcores=16)
    return pl.kernel(
        _combine_kernel,
        out_type=(
            jax.ShapeDtypeStruct((S, H), jnp.float32),
            jax.ShapeDtypeStruct((S, H), jnp.float32),
        ),
        mesh=mesh,
        compiler_params=pltpu.CompilerParams(needs_layout_passes=False),
        scratch_types=[
            pltpu.VMEM((CCHUNK,), jnp.int32),
            pltpu.VMEM((CCHUNK, H), jnp.float32),
            pltpu.SemaphoreType.DMA,
        ],
    )(pos0, pos1, y)


def _epilogue_kernel(w_ref, yg0_ref, yg1_ref, out_ref):
    out_ref[...] = (yg0_ref[...] * w_ref[:, 0:1] +
                    yg1_ref[...] * w_ref[:, 1:2])


def _run_epilogue(w, yg0, yg1):
    nrb = 8
    rb = S // nrb
    return pl.pallas_call(
        _epilogue_kernel,
        grid=(nrb,),
        in_specs=[
            pl.BlockSpec((rb, 2), lambda i: (i, 0)),
            pl.BlockSpec((rb, H), lambda i: (i, 0)),
            pl.BlockSpec((rb, H), lambda i: (i, 0)),
        ],
        out_specs=pl.BlockSpec((rb, H), lambda i: (i, 0)),
        out_shape=jax.ShapeDtypeStruct((S, H), jnp.float32),
    )(w, yg0, yg1)


@jax.jit
def kernel(hidden_states, gate_w, Wg, Wu, Wd):
    x = hidden_states.reshape(-1, H)
    router_logits, w, pos, be = _run_router(x, gate_w)
    pos0 = pos[:, 0]
    pos1 = pos[:, 1]
    xs = _run_dispatch(pos0, pos1, x)
    y = _run_ffn(be.reshape(NUM_BLOCKS), xs, Wg, Wu, Wd)
    yg0, yg1 = _run_combine(pos0, pos1, y)
    final = _run_epilogue(w, yg0, yg1)
    return final.reshape(hidden_states.shape), router_logits
